# baseline (speedup 1.0000x reference)
import functools

import jax
import jax.numpy as jnp
from jax import lax
from jax.experimental import pallas as pl
from jax.experimental.pallas import tpu as pltpu
from jax.experimental.pallas import tpu_sc as plsc

C = 16
E = 160000
K = 10
NB = 160
NW = 32
EP = 163840
NT = EP // (NW * NB)
NCH = EP // NB
EP2 = 174080
KNB = K * NB


def _sigmoid(z):
    return 1.0 / (1.0 + jnp.exp(-z))


def _bf16r(v):
    u = plsc.bitcast(v, jnp.uint32)
    r = u + jnp.uint32(0x7FFF) + ((u >> jnp.uint32(16)) & jnp.uint32(1))
    return plsc.bitcast(r & jnp.uint32(0xFFFF0000), jnp.float32)


@functools.partial(
    pl.kernel,
    out_type=jax.ShapeDtypeStruct((C, EP), jnp.float32),
    mesh=plsc.VectorSubcoreMesh(core_axis_name="c", subcore_axis_name="s"),
    compiler_params=pltpu.CompilerParams(
        needs_layout_passes=False, use_tc_tiling_on_sc=False
    ),
    scratch_types=[
        pltpu.VMEM((2, NB * K), jnp.int32),
        pltpu.VMEM((2, NB * K, C), jnp.float32),
        pltpu.VMEM((2, NB, C), jnp.float32),
        pltpu.VMEM((C * K * NB,), jnp.float32),
        pltpu.VMEM((C * NB,), jnp.float32),
        pltpu.VMEM((C, NB), jnp.float32),
        pltpu.VMEM((66 * C,), jnp.float32),
        pltpu.SemaphoreType.DMA,
        pltpu.SemaphoreType.DMA,
    ],
)
def _sc_attn(xr_h, meshf_h, wtab_h, out_h, idx_v, rows_v, xch_v,
             rowsT_v, xchT_v, outb_v, wtab_v, sem0, sem1):
    wid = lax.axis_index("s") * 2 + lax.axis_index("c")
    sems = (sem0, sem1)
    pltpu.sync_copy(wtab_h, wtab_v)
    iota = lax.iota(jnp.int32, C)
    iotaNB = iota * NB
    iotaKNB = iota * KNB
    one = jnp.full((C,), 1.0, jnp.float32)
    zero = jnp.full((C,), 0.0, jnp.float32)

    def issue(b, i):
        base = (wid + i * NW) * NB
        pltpu.sync_copy(meshf_h.at[pl.ds(base * K, NB * K)], idx_v.at[b])
        pltpu.sync_copy(xr_h.at[pl.ds(base, NB)], xch_v.at[b])
        for j in range(K):
            pltpu.async_copy(
                xr_h.at[idx_v.at[b].at[pl.ds(j * NB, NB)]],
                rows_v.at[b].at[pl.ds(j * NB, NB)],
                sems[b],
            )

    def wait(b):
        for j in range(K):
            pltpu.make_async_copy(
                xr_h.at[idx_v.at[b].at[pl.ds(j * NB, NB)]],
                rows_v.at[b].at[pl.ds(j * NB, NB)],
                sems[b],
            ).wait()

    def compute(b, i):
        base = (wid + i * NW) * NB

        def transpose_body(l, _):
            v = xch_v[b, l]
            plsc.store_scatter(xchT_v, [iotaNB + l], v)
            for k in range(K):
                r = rows_v[b, l * K + k]
                plsc.store_scatter(rowsT_v, [iotaKNB + (k * NB + l)], r)
            return _

        lax.fori_loop(0, NB, transpose_body, None)

        def group_body(g, _):
            g16 = g * C
            a = wtab_v[pl.ds(64 * C, C)]
            ce = wtab_v[pl.ds(65 * C, C)]
            sdot = [zero] * K
            udot = [zero] * K
            for c in range(C):
                xc = _bf16r(xchT_v[pl.ds(c * NB + g16, C)])
                a = a + wtab_v[pl.ds(c * C, C)] * xc
                ce = ce + wtab_v[pl.ds((32 + c) * C, C)] * xc
                wg = wtab_v[pl.ds((16 + c) * C, C)]
                rg = wtab_v[pl.ds((48 + c) * C, C)]
                for k in range(K):
                    gk = _bf16r(rowsT_v[pl.ds(c * KNB + k * NB + g16, C)])
                    sdot[k] = sdot[k] + wg * gk
                    udot[k] = udot[k] + rg * gk
            s = [a + sdot[k] for k in range(K)]
            u = [_sigmoid(ce + udot[k]) for k in range(K)]
            coef = []
            for k in range(K):
                cnt = zero
                for j in range(K):
                    if j == k:
                        continue
                    cnt = cnt + jnp.where(s[j] > s[k], one, zero)
                    if j < k:
                        cnt = cnt + jnp.where(s[j] == s[k], one, zero)
                coef.append(jnp.where(cnt < 4.0, u[k], zero))
            for c in range(C):
                acc = xchT_v[pl.ds(c * NB + g16, C)]
                for k in range(K):
                    gk = rowsT_v[pl.ds(c * KNB + k * NB + g16, C)]
                    acc = acc + coef[k] * gk
                outb_v[c, pl.ds(g16, C)] = acc
            return _

        lax.fori_loop(0, NB // C, group_body, None)
        pltpu.sync_copy(outb_v, out_h.at[:, pl.ds(base, NB)])

    issue(0, 0)

    def pair_body(t, _):
        i0 = 2 * t
        wait(0)
        issue(1, i0 + 1)
        compute(0, i0)
        wait(1)
        issue(0, i0 + 2)
        compute(1, i0 + 1)
        return _

    lax.fori_loop(0, NT // 2, pair_body, None)
    wait(0)


def kernel(x, mesh, conv_w, conv_b, rconv_w, rconv_b):
    xr = x.reshape(C, E).T
    xr = jnp.concatenate([xr, jnp.zeros((EP2 - E, C), jnp.float32)], axis=0)
    meshf = jnp.concatenate(
        [mesh.reshape(E * K).astype(jnp.int32),
         jnp.zeros(((EP2 - E) * K,), jnp.int32)]
    )

    def bf16r_host(v):
        u = lax.bitcast_convert_type(v, jnp.uint32)
        r = u + jnp.uint32(0x7FFF) + ((u >> jnp.uint32(16)) & jnp.uint32(1))
        return lax.bitcast_convert_type(r & jnp.uint32(0xFFFF0000), jnp.float32)

    cw = bf16r_host(conv_w.reshape(2 * C))
    rw = bf16r_host(rconv_w.reshape(2 * C))
    wtab = jnp.concatenate(
        [
            jnp.broadcast_to(cw[:, None], (2 * C, C)),
            jnp.broadcast_to(rw[:, None], (2 * C, C)),
            jnp.broadcast_to(conv_b.reshape(1, 1), (1, C)),
            jnp.broadcast_to(rconv_b.reshape(1, 1), (1, C)),
        ],
        axis=0,
    ).reshape(66 * C)
    out = _sc_attn(xr, meshf, wtab)
    return out[:, :E].reshape(1, C, E)


# speedup vs baseline: 1.6128x; 1.6128x over previous
import functools

import jax
import jax.numpy as jnp
from jax import lax
from jax.experimental import pallas as pl
from jax.experimental.pallas import tpu as pltpu
from jax.experimental.pallas import tpu_sc as plsc

C = 16
E = 160000
K = 10
NB = 128
NCHUNK = E // NB
NW = 32
KNB = K * NB


def _sigmoid(z):
    return 1.0 / (1.0 + jnp.exp(-z))


def _bf16r(v):
    u = plsc.bitcast(v, jnp.uint32)
    r = u + jnp.uint32(0x7FFF) + ((u >> jnp.uint32(16)) & jnp.uint32(1))
    return plsc.bitcast(r & jnp.uint32(0xFFFF0000), jnp.float32)


@functools.partial(
    pl.kernel,
    out_type=jax.ShapeDtypeStruct((C, E), jnp.float32),
    mesh=plsc.VectorSubcoreMesh(core_axis_name="c", subcore_axis_name="s"),
    compiler_params=pltpu.CompilerParams(
        needs_layout_passes=False, use_tc_tiling_on_sc=False
    ),
    scratch_types=[
        pltpu.VMEM((NB * K,), jnp.int32),
        pltpu.VMEM((NB * K, C), jnp.float32),
        pltpu.VMEM((C * K * NB,), jnp.float32),
        pltpu.VMEM((NB, C), jnp.float32),
        pltpu.VMEM((C * NB,), jnp.float32),
        pltpu.VMEM((C, NB), jnp.float32),
        pltpu.VMEM((66 * C,), jnp.float32),
        pltpu.SemaphoreType.DMA,
    ],
)
def _sc_attn(xr_h, meshf_h, wtab_h, out_h, idx_v, rows_v, rowsT_v, xch_v, xchT_v,
             outb_v, wtab_v, gsem):
    wid = lax.axis_index("s") * 2 + lax.axis_index("c")
    pltpu.sync_copy(wtab_h, wtab_v)
    nmine = (NCHUNK - wid + NW - 1) // NW
    iota = lax.iota(jnp.int32, C)
    iotaNB = iota * NB
    iotaKNB = iota * KNB
    one = jnp.full((C,), 1.0, jnp.float32)
    zero = jnp.full((C,), 0.0, jnp.float32)

    def chunk_body(i, _):
        chunk = wid + i * NW
        base = chunk * NB
        pltpu.sync_copy(meshf_h.at[pl.ds(base * K, NB * K)], idx_v)
        pltpu.sync_copy(xr_h.at[pl.ds(base, NB)], xch_v)
        descs = [
            pltpu.async_copy(
                xr_h.at[idx_v.at[pl.ds(j * NB, NB)]],
                rows_v.at[pl.ds(j * NB, NB)],
                gsem,
            )
            for j in range(K)
        ]
        for d in descs:
            d.wait()

        @plsc.parallel_loop(0, NB, unroll=2)
        def transpose_body(l):
            v = xch_v[l]
            plsc.store_scatter(xchT_v, [iotaNB + l], v)
            for k in range(K):
                r = rows_v[l * K + k]
                plsc.store_scatter(rowsT_v, [iotaKNB + (k * NB + l)], r)

        def group_body(g, _):
            g16 = g * C
            a = wtab_v[pl.ds(64 * C, C)]
            ce = wtab_v[pl.ds(65 * C, C)]
            sdot = [zero] * K
            udot = [zero] * K
            for c in range(C):
                xc = _bf16r(xchT_v[pl.ds(c * NB + g16, C)])
                a = a + wtab_v[pl.ds(c * C, C)] * xc
                ce = ce + wtab_v[pl.ds((32 + c) * C, C)] * xc
                wg = wtab_v[pl.ds((16 + c) * C, C)]
                rg = wtab_v[pl.ds((48 + c) * C, C)]
                for k in range(K):
                    gk = _bf16r(rowsT_v[pl.ds(c * KNB + k * NB + g16, C)])
                    sdot[k] = sdot[k] + wg * gk
                    udot[k] = udot[k] + rg * gk
            s = [a + sdot[k] for k in range(K)]
            u = [_sigmoid(ce + udot[k]) for k in range(K)]
            coef = []
            for k in range(K):
                cnt = zero
                for j in range(K):
                    if j == k:
                        continue
                    cnt = cnt + jnp.where(s[j] > s[k], one, zero)
                    if j < k:
                        cnt = cnt + jnp.where(s[j] == s[k], one, zero)
                coef.append(jnp.where(cnt < 4.0, u[k], zero))
            for c in range(C):
                acc = xchT_v[pl.ds(c * NB + g16, C)]
                for k in range(K):
                    gk = rowsT_v[pl.ds(c * KNB + k * NB + g16, C)]
                    acc = acc + coef[k] * gk
                outb_v[c, pl.ds(g16, C)] = acc
            return _

        lax.fori_loop(0, NB // C, group_body, None)
        pltpu.sync_copy(outb_v, out_h.at[:, pl.ds(base, NB)])
        return _

    lax.fori_loop(0, nmine, chunk_body, None)


def kernel(x, mesh, conv_w, conv_b, rconv_w, rconv_b):
    xr = x.reshape(C, E).T
    meshf = mesh.reshape(E * K).astype(jnp.int32)

    def bf16r_host(v):
        u = lax.bitcast_convert_type(v, jnp.uint32)
        r = u + jnp.uint32(0x7FFF) + ((u >> jnp.uint32(16)) & jnp.uint32(1))
        return lax.bitcast_convert_type(r & jnp.uint32(0xFFFF0000), jnp.float32)

    cw = bf16r_host(conv_w.reshape(2 * C))
    rw = bf16r_host(rconv_w.reshape(2 * C))
    wtab = jnp.concatenate(
        [
            jnp.broadcast_to(cw[:, None], (2 * C, C)),
            jnp.broadcast_to(rw[:, None], (2 * C, C)),
            jnp.broadcast_to(conv_b.reshape(1, 1), (1, C)),
            jnp.broadcast_to(rconv_b.reshape(1, 1), (1, C)),
        ],
        axis=0,
    ).reshape(66 * C)
    out = _sc_attn(xr, meshf, wtab)
    return out.reshape(1, C, E)
